# baseline (speedup 1.0000x reference)
import functools

import jax
import jax.numpy as jnp
from jax.experimental import pallas as pl
from jax.experimental.pallas import tpu as pltpu

N = 100000
E = 1600000
H = 32
ED = 4

B_E = 16000
B_N = 10000

BIG = 3.0e38


def _embed_body(x_ref, w_ref, b_ref, o_ref):
    o_ref[...] = x_ref[...] @ w_ref[...] + b_ref[...]


def _edge_mlp_body(xj_ref, ea_ref, w1a_ref, w1b_ref, b1_ref, w2_ref, b2_ref, m_ref):
    z = xj_ref[...] @ w1a_ref[...] + ea_ref[...] @ w1b_ref[...] + b1_ref[...]
    m_ref[...] = jnp.maximum(z, 0.0) @ w2_ref[...] + b2_ref[...]


def _node_mlp_body(h_ref, aggr_ref, w1a_ref, w1b_ref, b1_ref, w2_ref, b2_ref, o_ref):
    a = aggr_ref[...]
    a = jnp.where(a > 1e30, 0.0, a)
    z = h_ref[...] @ w1a_ref[...] + a @ w1b_ref[...] + b1_ref[...]
    o_ref[...] = jnp.maximum(z, 0.0) @ w2_ref[...] + b2_ref[...]


def _full(shape):
    return pl.BlockSpec(shape, lambda i: (0,) * len(shape))


def _embed(x, w, b):
    return pl.pallas_call(
        _embed_body,
        grid=(N // B_N,),
        in_specs=[pl.BlockSpec((B_N, 1), lambda i: (i, 0)),
                  _full((1, H)), _full((1, H))],
        out_specs=pl.BlockSpec((B_N, H), lambda i: (i, 0)),
        out_shape=jax.ShapeDtypeStruct((N, H), jnp.float32),
    )(x, w, b)


def _edge_mlp(xj, ea, w1a, w1b, b1, w2, b2):
    return pl.pallas_call(
        _edge_mlp_body,
        grid=(E // B_E,),
        in_specs=[pl.BlockSpec((B_E, H), lambda i: (i, 0)),
                  pl.BlockSpec((B_E, ED), lambda i: (i, 0)),
                  _full((H, H)), _full((ED, H)), _full((1, H)),
                  _full((H, H)), _full((1, H))],
        out_specs=pl.BlockSpec((B_E, H), lambda i: (i, 0)),
        out_shape=jax.ShapeDtypeStruct((E, H), jnp.float32),
    )(xj, ea, w1a, w1b, b1, w2, b2)


def _node_mlp(h, aggr, w1a, w1b, b1, w2, b2, out_dim):
    return pl.pallas_call(
        _node_mlp_body,
        grid=(N // B_N,),
        in_specs=[pl.BlockSpec((B_N, H), lambda i: (i, 0)),
                  pl.BlockSpec((B_N, H), lambda i: (i, 0)),
                  _full((H, H)), _full((H, H)), _full((1, H)),
                  _full((H, out_dim)), _full((1, out_dim))],
        out_specs=pl.BlockSpec((B_N, out_dim), lambda i: (i, 0)),
        out_shape=jax.ShapeDtypeStruct((N, out_dim), jnp.float32),
    )(h, aggr, w1a, w1b, b1, w2, b2)


def kernel(x, edge_index, edge_attr, emb_W, emb_b, msg_W1, msg_b1, msg_W2, msg_b2,
           upd_W1, upd_b1, upd_W2, upd_b2, fc_W, fc_b):
    L = msg_W1.shape[0]
    src = edge_index[0]
    dst = edge_index[1]

    h = _embed(x, emb_W, emb_b.reshape(1, H))

    w2_last = upd_W2[L - 1] @ fc_W
    b2_last = (upd_b2[L - 1] @ fc_W + fc_b).reshape(1, 1)

    for l in range(L):
        xj = jnp.take(h, src, axis=0)
        m = _edge_mlp(xj, edge_attr,
                      msg_W1[l, :H], msg_W1[l, H:], msg_b1[l].reshape(1, H),
                      msg_W2[l], msg_b2[l].reshape(1, H))
        aggr = jnp.full((N, H), BIG, jnp.float32).at[dst].min(m)
        last = l == L - 1
        h = _node_mlp(h, aggr,
                      upd_W1[l, :H], upd_W1[l, H:], upd_b1[l].reshape(1, H),
                      w2_last if last else upd_W2[l],
                      b2_last if last else upd_b2[l].reshape(1, H),
                      1 if last else H)
    return h[:, 0]


# speedup vs baseline: 1.6720x; 1.6720x over previous
import functools

import jax
import jax.numpy as jnp
from jax import lax
from jax.experimental import pallas as pl
from jax.experimental.pallas import tpu as pltpu
from jax.experimental.pallas import tpu_sc as plsc

N = 100000
E = 1600000
H = 32
ED = 4

B_E = 16000
B_N = 10000

BIG = 3.0e38


def _embed_body(x_ref, w_ref, b_ref, o_ref):
    o_ref[...] = x_ref[...] @ w_ref[...] + b_ref[...]


def _edge_mlp_body(xj_ref, ea_ref, w1a_ref, w1b_ref, b1_ref, w2_ref, b2_ref, m_ref):
    z = xj_ref[...] @ w1a_ref[...] + ea_ref[...] @ w1b_ref[...] + b1_ref[...]
    m_ref[...] = jnp.maximum(z, 0.0) @ w2_ref[...] + b2_ref[...]


def _node_mlp_body(h_ref, aggr_ref, w1a_ref, w1b_ref, b1_ref, w2_ref, b2_ref, o_ref):
    a = aggr_ref[...]
    a = jnp.where(a > 1e30, 0.0, a)
    z = h_ref[...] @ w1a_ref[...] + a @ w1b_ref[...] + b1_ref[...]
    o_ref[...] = jnp.maximum(z, 0.0) @ w2_ref[...] + b2_ref[...]


_NW = 32
_GC = 2000


def _make_sc_gather():
    per_w = E // _NW
    chunks = per_w // _GC

    mesh = plsc.VectorSubcoreMesh(core_axis_name="c", subcore_axis_name="s")

    @functools.partial(
        pl.kernel, mesh=mesh,
        out_type=jax.ShapeDtypeStruct((E, H), jnp.float32),
        compiler_params=pltpu.CompilerParams(use_tc_tiling_on_sc=False),
        scratch_types=[
            pltpu.VMEM((_GC,), jnp.int32),
            pltpu.VMEM((_GC, H), jnp.float32),
            pltpu.SemaphoreType.DMA,
        ],
    )
    def gather_k(table_hbm, idx_hbm, out_hbm, idx_v, rows_v, sem):
        wid = lax.axis_index("s") * 2 + lax.axis_index("c")
        base = wid * per_w

        def body(i, carry):
            off = base + i * _GC
            pltpu.sync_copy(idx_hbm.at[pl.ds(off, _GC)], idx_v)
            pltpu.async_copy(table_hbm.at[idx_v], rows_v, sem).wait()
            pltpu.sync_copy(rows_v, out_hbm.at[pl.ds(off, _GC)])
            return carry

        lax.fori_loop(0, chunks, body, 0)

    return gather_k


_sc_gather = _make_sc_gather()


def _full(shape):
    return pl.BlockSpec(shape, lambda i: (0,) * len(shape))


def _embed(x, w, b):
    return pl.pallas_call(
        _embed_body,
        grid=(N // B_N,),
        in_specs=[pl.BlockSpec((B_N, 1), lambda i: (i, 0)),
                  _full((1, H)), _full((1, H))],
        out_specs=pl.BlockSpec((B_N, H), lambda i: (i, 0)),
        out_shape=jax.ShapeDtypeStruct((N, H), jnp.float32),
    )(x, w, b)


def _edge_mlp(xj, ea, w1a, w1b, b1, w2, b2):
    return pl.pallas_call(
        _edge_mlp_body,
        grid=(E // B_E,),
        in_specs=[pl.BlockSpec((B_E, H), lambda i: (i, 0)),
                  pl.BlockSpec((B_E, ED), lambda i: (i, 0)),
                  _full((H, H)), _full((ED, H)), _full((1, H)),
                  _full((H, H)), _full((1, H))],
        out_specs=pl.BlockSpec((B_E, H), lambda i: (i, 0)),
        out_shape=jax.ShapeDtypeStruct((E, H), jnp.float32),
    )(xj, ea, w1a, w1b, b1, w2, b2)


def _node_mlp(h, aggr, w1a, w1b, b1, w2, b2, out_dim):
    return pl.pallas_call(
        _node_mlp_body,
        grid=(N // B_N,),
        in_specs=[pl.BlockSpec((B_N, H), lambda i: (i, 0)),
                  pl.BlockSpec((B_N, H), lambda i: (i, 0)),
                  _full((H, H)), _full((H, H)), _full((1, H)),
                  _full((H, out_dim)), _full((1, out_dim))],
        out_specs=pl.BlockSpec((B_N, out_dim), lambda i: (i, 0)),
        out_shape=jax.ShapeDtypeStruct((N, out_dim), jnp.float32),
    )(h, aggr, w1a, w1b, b1, w2, b2)


def kernel(x, edge_index, edge_attr, emb_W, emb_b, msg_W1, msg_b1, msg_W2, msg_b2,
           upd_W1, upd_b1, upd_W2, upd_b2, fc_W, fc_b):
    L = msg_W1.shape[0]
    src = edge_index[0]
    dst = edge_index[1]

    h = _embed(x, emb_W, emb_b.reshape(1, H))

    w2_last = upd_W2[L - 1] @ fc_W
    b2_last = (upd_b2[L - 1] @ fc_W + fc_b).reshape(1, 1)

    for l in range(L):
        xj = _sc_gather(h, src)
        m = _edge_mlp(xj, edge_attr,
                      msg_W1[l, :H], msg_W1[l, H:], msg_b1[l].reshape(1, H),
                      msg_W2[l], msg_b2[l].reshape(1, H))
        aggr = jnp.full((N, H), BIG, jnp.float32).at[dst].min(m)
        last = l == L - 1
        h = _node_mlp(h, aggr,
                      upd_W1[l, :H], upd_W1[l, H:], upd_b1[l].reshape(1, H),
                      w2_last if last else upd_W2[l],
                      b2_last if last else upd_b2[l].reshape(1, H),
                      1 if last else H)
    return h[:, 0]


# speedup vs baseline: 3.0916x; 1.8490x over previous
import functools

import jax
import jax.numpy as jnp
from jax import lax
from jax.experimental import pallas as pl
from jax.experimental.pallas import tpu as pltpu
from jax.experimental.pallas import tpu_sc as plsc

N = 100000
E = 1600000
H = 32
ED = 4

B_E = 16000
B_N = 10000

BIG = 3.0e38


def _embed_body(x_ref, w_ref, b_ref, o_ref):
    o_ref[...] = x_ref[...] @ w_ref[...] + b_ref[...]


def _edge_mlp_body(xj_ref, ea_ref, w1a_ref, w1b_ref, b1_ref, w2_ref, b2_ref, m_ref):
    z = xj_ref[...] @ w1a_ref[...] + ea_ref[...] @ w1b_ref[...] + b1_ref[...]
    m_ref[...] = jnp.maximum(z, 0.0) @ w2_ref[...] + b2_ref[...]


def _node_mlp_body(h_ref, aggr_ref, w1a_ref, w1b_ref, b1_ref, w2_ref, b2_ref, o_ref):
    a = aggr_ref[...]
    a = jnp.where(a > 1e30, 0.0, a)
    z = h_ref[...] @ w1a_ref[...] + a @ w1b_ref[...] + b1_ref[...]
    o_ref[...] = jnp.maximum(z, 0.0) @ w2_ref[...] + b2_ref[...]


_NW = 32
_GC = 2000


def _make_sc_gather():
    per_w = E // _NW
    chunks = per_w // _GC

    mesh = plsc.VectorSubcoreMesh(core_axis_name="c", subcore_axis_name="s")

    @functools.partial(
        pl.kernel, mesh=mesh,
        out_type=jax.ShapeDtypeStruct((E, H), jnp.float32),
        compiler_params=pltpu.CompilerParams(use_tc_tiling_on_sc=False, needs_layout_passes=False),
        scratch_types=[
            pltpu.VMEM((_GC,), jnp.int32),
            pltpu.VMEM((_GC, H), jnp.float32),
            pltpu.SemaphoreType.DMA,
        ],
    )
    def gather_k(table_hbm, idx_hbm, out_hbm, idx_v, rows_v, sem):
        wid = lax.axis_index("s") * 2 + lax.axis_index("c")
        base = wid * per_w

        def body(i, carry):
            off = base + i * _GC
            pltpu.sync_copy(idx_hbm.at[pl.ds(pl.multiple_of(off, 8), _GC)], idx_v)
            pltpu.async_copy(table_hbm.at[idx_v], rows_v, sem).wait()
            pltpu.sync_copy(rows_v, out_hbm.at[pl.ds(pl.multiple_of(off, 8), _GC)])
            return carry

        lax.fori_loop(0, chunks, body, 0)

    return gather_k


_sc_gather = _make_sc_gather()


NB = N // _NW
_SUB = 16
ECAP = E + _NW * _NW * _SUB + 1024
_C = 512
_HC = 2000
_PER_W = E // _NW
_INV_NB = 1.0 / NB


def _bucket_of(dv):
    return ((dv.astype(jnp.float32) + 0.5) * _INV_NB).astype(jnp.int32)


def _iota16():
    return lax.iota(jnp.int32, 16)


def _dyn_lane(v0, v1, idx):
    i = _iota16()
    lo = jnp.sum(jnp.where(i == idx, v0, 0))
    hi = jnp.sum(jnp.where(i == idx - 16, v1, 0))
    return lo + hi


def _region_layout(cnt_ref, my_p):
    z = jnp.zeros((16,), jnp.int32)

    def body(pp, carry):
        t0, t1, p0, p1 = carry
        r0 = cnt_ref[pp, pl.ds(0, 16)]
        r1 = cnt_ref[pp, pl.ds(16, 16)]
        s0 = jnp.bitwise_and(r0 + (_SUB - 1), -_SUB)
        s1 = jnp.bitwise_and(r1 + (_SUB - 1), -_SUB)
        m = pp < my_p
        return (t0 + s0, t1 + s1,
                p0 + jnp.where(m, s0, 0), p1 + jnp.where(m, s1, 0))

    t0, t1, p0, p1 = lax.fori_loop(0, _NW, body, (z, z, z, z))
    c0 = plsc.cumsum(t0)
    b0 = c0 - t0
    all0 = c0[15]
    c1 = plsc.cumsum(t1)
    b1 = c1 - t1 + all0
    return b0, b1, p0, p1, t0, t1


def _make_sc_hist():
    nch = _PER_W // _HC
    mesh = plsc.VectorSubcoreMesh(core_axis_name="c", subcore_axis_name="s")

    @functools.partial(
        pl.kernel, mesh=mesh,
        out_type=jax.ShapeDtypeStruct((_NW, _NW), jnp.int32),
        compiler_params=pltpu.CompilerParams(use_tc_tiling_on_sc=False, needs_layout_passes=False),
        scratch_types=[pltpu.VMEM((_HC,), jnp.int32),
                       pltpu.VMEM((_NW,), jnp.int32)],
    )
    def hist_k(dst_hbm, cnt_hbm, dst_v, cnt_v):
        wid = lax.axis_index("s") * 2 + lax.axis_index("c")
        base = wid * _PER_W
        it = _iota16()
        z = jnp.zeros((16,), jnp.int32)

        def chunk(k, carry):
            pltpu.sync_copy(dst_hbm.at[pl.ds(pl.multiple_of(base + k * _HC, 8), _HC)], dst_v)

            def grp(g, cc):
                d0, d1 = cc
                bv = _bucket_of(dst_v[pl.ds(g * 16, 16)])
                for j in range(16):
                    bj = bv[j]
                    d0 = d0 + jnp.where(it == bj, 1, 0)
                    d1 = d1 + jnp.where(it == bj - 16, 1, 0)
                return d0, d1

            return lax.fori_loop(0, _HC // 16, grp, carry)

        c0, c1 = lax.fori_loop(0, nch, chunk, (z, z))
        cnt_v[pl.ds(0, 16)] = c0
        cnt_v[pl.ds(16, 16)] = c1
        pltpu.sync_copy(cnt_v, cnt_hbm.at[wid])

    return hist_k


def _make_sc_bin():
    nch = _PER_W // _HC
    mesh = plsc.VectorSubcoreMesh(core_axis_name="c", subcore_axis_name="s")

    @functools.partial(
        pl.kernel, mesh=mesh,
        out_type=jax.ShapeDtypeStruct((ECAP,), jnp.int32),
        compiler_params=pltpu.CompilerParams(use_tc_tiling_on_sc=False, needs_layout_passes=False),
        scratch_types=[pltpu.VMEM((_HC,), jnp.int32),
                       pltpu.VMEM((_NW, _NW), jnp.int32),
                       pltpu.VMEM((_NW * 2 * _SUB,), jnp.int32),
                       pltpu.SMEM((64,), jnp.int32)],
    )
    def bin_k(dst_hbm, cnt_hbm, beid_hbm, dst_v, cnt_v, fbuf, sm):
        wid = lax.axis_index("s") * 2 + lax.axis_index("c")
        base = wid * _PER_W
        pltpu.sync_copy(cnt_hbm, cnt_v)
        b0, b1, p0, p1, t0, t1 = _region_layout(cnt_v, wid)
        g0 = b0 + p0
        g1 = b1 + p1
        it = _iota16()
        for j in range(16):
            sm[j] = 0
            sm[16 + j] = 0
            sm[32 + j] = g0[j]
            sm[48 + j] = g1[j]

        def chunk(k, carry):
            pltpu.sync_copy(dst_hbm.at[pl.ds(pl.multiple_of(base + k * _HC, 8), _HC)], dst_v)

            def grp(g, cc):
                bv = _bucket_of(dst_v[pl.ds(g * 16, 16)])
                eids = base + k * _HC + g * 16 + it
                slots = jnp.zeros((16,), jnp.int32)
                post = []
                for j in range(16):
                    bj = bv[j]
                    w = sm[bj]
                    slots = jnp.where(it == j,
                                      bj * 32 + jnp.bitwise_and(w, 31), slots)
                    sm[bj] = w + 1
                    post.append((bj, w + 1))
                plsc.store_scatter(fbuf, [slots], eids)
                for bj, w1 in post:
                    @pl.when(jnp.bitwise_and(w1, 15) == 0)
                    def _flush(bj=bj, w1=w1):
                        gb = sm[32 + bj]
                        half = jnp.bitwise_and(w1 - 16, 31)
                        pltpu.sync_copy(
                            fbuf.at[pl.ds(pl.multiple_of(bj * 32 + half, 8), 16)],
                            beid_hbm.at[pl.ds(pl.multiple_of(gb + w1 - 16, 8), 16)])
                return cc

            return lax.fori_loop(0, _HC // 16, grp, carry)

        lax.fori_loop(0, nch, chunk, 0)

        for b in range(_NW):
            w = sm[b]
            r = jnp.bitwise_and(w, 15)

            @pl.when(r != 0)
            def _tail(b=b, w=w, r=r):
                half = jnp.bitwise_and(w - r, 31)
                hv = fbuf[pl.ds(pl.multiple_of(b * 32 + half, 8), 16)]
                last = jnp.sum(jnp.where(it == r - 1, hv, 0))
                fbuf[pl.ds(pl.multiple_of(b * 32 + half, 8), 16)] = jnp.where(it < r, hv, last)
                gb = sm[32 + b]
                pltpu.sync_copy(fbuf.at[pl.ds(pl.multiple_of(b * 32 + half, 8), 16)],
                                beid_hbm.at[pl.ds(pl.multiple_of(gb + w - r, 8), 16)])

        total = b1[15] + t1[15]

        @pl.when(wid == _NW - 1)
        def _slack():
            zero = jnp.zeros((16,), jnp.int32)

            def zb(i, c):
                fbuf[pl.ds(i * 16, 16)] = zero
                return c

            lax.fori_loop(0, 64, zb, 0)
            pltpu.sync_copy(
                fbuf.at[pl.ds(0, _NW * 2 * _SUB)],
                beid_hbm.at[pl.ds(pl.multiple_of(total, 8), _NW * 2 * _SUB)])

    return bin_k


def _make_sc_scatter():
    acc_n = N * H // _NW
    mesh = plsc.VectorSubcoreMesh(core_axis_name="c", subcore_axis_name="s")

    @functools.partial(
        pl.kernel, mesh=mesh,
        out_type=jax.ShapeDtypeStruct((N * H,), jnp.float32),
        compiler_params=pltpu.CompilerParams(use_tc_tiling_on_sc=False, needs_layout_passes=False),
        scratch_types=[pltpu.VMEM((acc_n,), jnp.float32),
                       pltpu.VMEM((_C,), jnp.int32),
                       pltpu.VMEM((_C, H), jnp.float32),
                       pltpu.VMEM((_C,), jnp.int32),
                       pltpu.VMEM((_NW, _NW), jnp.int32),
                       pltpu.SemaphoreType.DMA,
                       pltpu.SemaphoreType.DMA],
    )
    def scatter_k(m_hbm, dst_hbm, beid_hbm, cnt_hbm, aggr_hbm,
                  acc_v, eid_v, rows_v, dstc_v, cnt_v, sem, sem2):
        wid = lax.axis_index("s") * 2 + lax.axis_index("c")
        pltpu.sync_copy(cnt_hbm, cnt_v)
        b0, b1, _, _, t0, t1 = _region_layout(cnt_v, wid)
        rbase = _dyn_lane(b0, b1, wid)
        cap = _dyn_lane(t0, t1, wid)
        n0 = wid * NB

        big = jnp.full((16,), BIG, jnp.float32)

        def init(i, c):
            acc_v[pl.ds(i * 16, 16)] = big
            return c

        lax.fori_loop(0, acc_n // 16, init, 0)

        def chunk(k, c):
            off = rbase + k * _C
            pltpu.sync_copy(beid_hbm.at[pl.ds(pl.multiple_of(off, 8), _C)], eid_v)

            def cl(i, c):
                v = eid_v[pl.ds(i * 16, 16)]
                eid_v[pl.ds(i * 16, 16)] = jnp.clip(v, 0, E - 1)
                return c

            lax.fori_loop(0, _C // 16, cl, 0)
            cp1 = pltpu.async_copy(m_hbm.at[eid_v], rows_v, sem)
            cp2 = pltpu.async_copy(dst_hbm.at[eid_v], dstc_v, sem2)
            cp1.wait()
            cp2.wait()
            valid = jnp.minimum(cap - k * _C, _C)

            def grp(g, cc):
                dl = jnp.clip(dstc_v[pl.ds(g * 16, 16)] - n0, 0, NB - 1) * H
                for j in range(16):
                    o = dl[j]
                    gi = g * 16 + j
                    r0 = rows_v[gi, pl.ds(0, 16)]
                    r1 = rows_v[gi, pl.ds(16, 16)]
                    a0 = acc_v[pl.ds(o, 16)]
                    a1 = acc_v[pl.ds(o + 16, 16)]
                    acc_v[pl.ds(o, 16)] = jnp.minimum(a0, r0)
                    acc_v[pl.ds(o + 16, 16)] = jnp.minimum(a1, r1)
                return cc

            lax.fori_loop(0, valid // 16, grp, 0)
            return c

        nch = (cap + _C - 1) // _C
        lax.fori_loop(0, nch, chunk, 0)
        pltpu.sync_copy(acc_v, aggr_hbm.at[pl.ds(pl.multiple_of(wid * acc_n, 8), acc_n)])

    return scatter_k


_sc_hist = _make_sc_hist()
_sc_bin = _make_sc_bin()
_sc_scatter = _make_sc_scatter()


def _full(shape):
    return pl.BlockSpec(shape, lambda i: (0,) * len(shape))


def _embed(x, w, b):
    return pl.pallas_call(
        _embed_body,
        grid=(N // B_N,),
        in_specs=[pl.BlockSpec((B_N, 1), lambda i: (i, 0)),
                  _full((1, H)), _full((1, H))],
        out_specs=pl.BlockSpec((B_N, H), lambda i: (i, 0)),
        out_shape=jax.ShapeDtypeStruct((N, H), jnp.float32),
    )(x, w, b)


def _edge_mlp(xj, ea, w1a, w1b, b1, w2, b2):
    return pl.pallas_call(
        _edge_mlp_body,
        grid=(E // B_E,),
        in_specs=[pl.BlockSpec((B_E, H), lambda i: (i, 0)),
                  pl.BlockSpec((B_E, ED), lambda i: (i, 0)),
                  _full((H, H)), _full((ED, H)), _full((1, H)),
                  _full((H, H)), _full((1, H))],
        out_specs=pl.BlockSpec((B_E, H), lambda i: (i, 0)),
        out_shape=jax.ShapeDtypeStruct((E, H), jnp.float32),
    )(xj, ea, w1a, w1b, b1, w2, b2)


def _node_mlp(h, aggr, w1a, w1b, b1, w2, b2, out_dim):
    return pl.pallas_call(
        _node_mlp_body,
        grid=(N // B_N,),
        in_specs=[pl.BlockSpec((B_N, H), lambda i: (i, 0)),
                  pl.BlockSpec((B_N, H), lambda i: (i, 0)),
                  _full((H, H)), _full((H, H)), _full((1, H)),
                  _full((H, out_dim)), _full((1, out_dim))],
        out_specs=pl.BlockSpec((B_N, out_dim), lambda i: (i, 0)),
        out_shape=jax.ShapeDtypeStruct((N, out_dim), jnp.float32),
    )(h, aggr, w1a, w1b, b1, w2, b2)


def kernel(x, edge_index, edge_attr, emb_W, emb_b, msg_W1, msg_b1, msg_W2, msg_b2,
           upd_W1, upd_b1, upd_W2, upd_b2, fc_W, fc_b):
    L = msg_W1.shape[0]
    src = edge_index[0]
    dst = edge_index[1]

    cnt = _sc_hist(dst)
    beid = _sc_bin(dst, cnt)

    h = _embed(x, emb_W, emb_b.reshape(1, H))

    w2_last = upd_W2[L - 1] @ fc_W
    b2_last = (upd_b2[L - 1] @ fc_W + fc_b).reshape(1, 1)

    for l in range(L):
        xj = _sc_gather(h, src)
        m = _edge_mlp(xj, edge_attr,
                      msg_W1[l, :H], msg_W1[l, H:], msg_b1[l].reshape(1, H),
                      msg_W2[l], msg_b2[l].reshape(1, H))
        aggr = _sc_scatter(m, dst, beid, cnt).reshape(N, H)
        last = l == L - 1
        h = _node_mlp(h, aggr,
                      upd_W1[l, :H], upd_W1[l, H:], upd_b1[l].reshape(1, H),
                      w2_last if last else upd_W2[l],
                      b2_last if last else upd_b2[l].reshape(1, H),
                      1 if last else H)
    return h[:, 0]


# speedup vs baseline: 3.1124x; 1.0067x over previous
import functools

import jax
import jax.numpy as jnp
from jax import lax
from jax.experimental import pallas as pl
from jax.experimental.pallas import tpu as pltpu
from jax.experimental.pallas import tpu_sc as plsc

N = 100000
E = 1600000
H = 32
ED = 4

B_E = 16000
B_N = 10000

BIG = 3.0e38


def _embed_body(x_ref, w_ref, b_ref, o_ref):
    o_ref[...] = x_ref[...] @ w_ref[...] + b_ref[...]


def _edge_mlp_body(xj_ref, ea_ref, w1a_ref, w1b_ref, b1_ref, w2_ref, b2_ref, m_ref):
    z = xj_ref[...] @ w1a_ref[...] + ea_ref[...] @ w1b_ref[...] + b1_ref[...]
    m_ref[...] = jnp.maximum(z, 0.0) @ w2_ref[...] + b2_ref[...]


def _node_mlp_body(h_ref, alo_ref, ahi_ref, w1a_ref, w1bl_ref, w1bh_ref,
                   b1_ref, w2_ref, b2_ref, o_ref):
    alo = alo_ref[...]
    alo = jnp.where(alo > 1e30, 0.0, alo)
    ahi = ahi_ref[...]
    ahi = jnp.where(ahi > 1e30, 0.0, ahi)
    z = (h_ref[...] @ w1a_ref[...] + alo @ w1bl_ref[...]
         + ahi @ w1bh_ref[...] + b1_ref[...])
    o_ref[...] = jnp.maximum(z, 0.0) @ w2_ref[...] + b2_ref[...]


_NW = 32
_GC = 1000


def _make_sc_gather():
    per_w = E // _NW
    chunks = per_w // _GC

    mesh = plsc.VectorSubcoreMesh(core_axis_name="c", subcore_axis_name="s")

    @functools.partial(
        pl.kernel, mesh=mesh,
        out_type=jax.ShapeDtypeStruct((E, H), jnp.float32),
        compiler_params=pltpu.CompilerParams(use_tc_tiling_on_sc=False, needs_layout_passes=False),
        scratch_types=[
            pltpu.VMEM((2, _GC), jnp.int32),
            pltpu.VMEM((2, _GC, H), jnp.float32),
            pltpu.SemaphoreType.DMA,
            pltpu.SemaphoreType.DMA,
            pltpu.SemaphoreType.DMA,
            pltpu.SemaphoreType.DMA,
        ],
    )
    def gather_k(table_hbm, idx_hbm, out_hbm, idx_v, rows_v, s0, s1, o0, o1):
        wid = lax.axis_index("s") * 2 + lax.axis_index("c")
        base = wid * per_w
        gsem = [s0, s1]
        osem = [o0, o1]
        out_cp = [None, None]
        for i in range(chunks):
            b = i & 1
            off = base + i * _GC
            if out_cp[b] is not None:
                out_cp[b].wait()
            pltpu.sync_copy(idx_hbm.at[pl.ds(pl.multiple_of(off, 8), _GC)],
                            idx_v.at[b])
            pltpu.async_copy(table_hbm.at[idx_v.at[b]], rows_v.at[b],
                             gsem[b]).wait()
            out_cp[b] = pltpu.async_copy(
                rows_v.at[b],
                out_hbm.at[pl.ds(pl.multiple_of(off, 8), _GC)], osem[b])
        out_cp[0].wait()
        out_cp[1].wait()

    return gather_k


_sc_gather = _make_sc_gather()


NB = N // _NW
_SUB = 16
ECAP = E + _NW * _NW * _SUB + 2048
_C = 384
_HC = 2000
_PER_W = E // _NW
_INV_NB = 1.0 / NB


def _bucket_of(dv):
    return ((dv.astype(jnp.float32) + 0.5) * _INV_NB).astype(jnp.int32)


def _iota16():
    return lax.iota(jnp.int32, 16)


def _dyn_lane(v0, v1, idx):
    i = _iota16()
    lo = jnp.sum(jnp.where(i == idx, v0, 0))
    hi = jnp.sum(jnp.where(i == idx - 16, v1, 0))
    return lo + hi


def _region_layout(cnt_ref, my_p):
    z = jnp.zeros((16,), jnp.int32)

    def body(pp, carry):
        t0, t1, p0, p1 = carry
        r0 = cnt_ref[pp, pl.ds(0, 16)]
        r1 = cnt_ref[pp, pl.ds(16, 16)]
        s0 = jnp.bitwise_and(r0 + (_SUB - 1), -_SUB)
        s1 = jnp.bitwise_and(r1 + (_SUB - 1), -_SUB)
        m = pp < my_p
        return (t0 + s0, t1 + s1,
                p0 + jnp.where(m, s0, 0), p1 + jnp.where(m, s1, 0))

    t0, t1, p0, p1 = lax.fori_loop(0, _NW, body, (z, z, z, z))
    c0 = plsc.cumsum(t0)
    b0 = c0 - t0
    all0 = c0[15]
    c1 = plsc.cumsum(t1)
    b1 = c1 - t1 + all0
    return b0, b1, p0, p1, t0, t1


def _make_sc_hist():
    nch = _PER_W // _HC
    mesh = plsc.VectorSubcoreMesh(core_axis_name="c", subcore_axis_name="s")

    @functools.partial(
        pl.kernel, mesh=mesh,
        out_type=jax.ShapeDtypeStruct((_NW, _NW), jnp.int32),
        compiler_params=pltpu.CompilerParams(use_tc_tiling_on_sc=False, needs_layout_passes=False),
        scratch_types=[pltpu.VMEM((_HC,), jnp.int32),
                       pltpu.VMEM((_NW,), jnp.int32)],
    )
    def hist_k(dst_hbm, cnt_hbm, dst_v, cnt_v):
        wid = lax.axis_index("s") * 2 + lax.axis_index("c")
        base = wid * _PER_W
        it = _iota16()
        z = jnp.zeros((16,), jnp.int32)

        def chunk(k, carry):
            pltpu.sync_copy(dst_hbm.at[pl.ds(pl.multiple_of(base + k * _HC, 8), _HC)], dst_v)

            def grp(g, cc):
                d0, d1 = cc
                bv = _bucket_of(dst_v[pl.ds(g * 16, 16)])
                for j in range(16):
                    bj = bv[j]
                    d0 = d0 + jnp.where(it == bj, 1, 0)
                    d1 = d1 + jnp.where(it == bj - 16, 1, 0)
                return d0, d1

            return lax.fori_loop(0, _HC // 16, grp, carry)

        c0, c1 = lax.fori_loop(0, nch, chunk, (z, z))
        cnt_v[pl.ds(0, 16)] = c0
        cnt_v[pl.ds(16, 16)] = c1
        pltpu.sync_copy(cnt_v, cnt_hbm.at[wid])

    return hist_k


def _make_sc_bin():
    nch = _PER_W // _HC
    mesh = plsc.VectorSubcoreMesh(core_axis_name="c", subcore_axis_name="s")

    @functools.partial(
        pl.kernel, mesh=mesh,
        out_type=(jax.ShapeDtypeStruct((ECAP,), jnp.int32),
                  jax.ShapeDtypeStruct((ECAP,), jnp.int32)),
        compiler_params=pltpu.CompilerParams(use_tc_tiling_on_sc=False, needs_layout_passes=False),
        scratch_types=[pltpu.VMEM((_HC,), jnp.int32),
                       pltpu.VMEM((_NW, _NW), jnp.int32),
                       pltpu.VMEM((_NW * 2 * _SUB,), jnp.int32),
                       pltpu.VMEM((_NW * 2 * _SUB,), jnp.int32),
                       pltpu.SMEM((64,), jnp.int32)],
    )
    def bin_k(dst_hbm, cnt_hbm, beid_hbm, bofs_hbm, dst_v, cnt_v, fbuf, fbuf2, sm):
        wid = lax.axis_index("s") * 2 + lax.axis_index("c")
        base = wid * _PER_W
        pltpu.sync_copy(cnt_hbm, cnt_v)
        b0, b1, p0, p1, t0, t1 = _region_layout(cnt_v, wid)
        g0 = b0 + p0
        g1 = b1 + p1
        it = _iota16()
        for j in range(16):
            sm[j] = 0
            sm[16 + j] = 0
            sm[32 + j] = g0[j]
            sm[48 + j] = g1[j]

        def chunk(k, carry):
            pltpu.sync_copy(dst_hbm.at[pl.ds(pl.multiple_of(base + k * _HC, 8), _HC)], dst_v)

            def grp(g, cc):
                dv = dst_v[pl.ds(g * 16, 16)]
                bv = _bucket_of(dv)
                ofsv = (dv - bv * NB) * (H // 2)
                eids = base + k * _HC + g * 16 + it
                slots = jnp.zeros((16,), jnp.int32)
                post = []
                for j in range(16):
                    bj = bv[j]
                    w = sm[bj]
                    slots = jnp.where(it == j,
                                      bj * 32 + jnp.bitwise_and(w, 31), slots)
                    sm[bj] = w + 1
                    post.append((bj, w + 1))
                plsc.store_scatter(fbuf, [slots], eids)
                plsc.store_scatter(fbuf2, [slots], ofsv)
                for bj, w1 in post:
                    @pl.when(jnp.bitwise_and(w1, 15) == 0)
                    def _flush(bj=bj, w1=w1):
                        gb = sm[32 + bj]
                        half = jnp.bitwise_and(w1 - 16, 31)
                        pltpu.sync_copy(
                            fbuf.at[pl.ds(pl.multiple_of(bj * 32 + half, 8), 16)],
                            beid_hbm.at[pl.ds(pl.multiple_of(gb + w1 - 16, 8), 16)])
                        pltpu.sync_copy(
                            fbuf2.at[pl.ds(pl.multiple_of(bj * 32 + half, 8), 16)],
                            bofs_hbm.at[pl.ds(pl.multiple_of(gb + w1 - 16, 8), 16)])
                return cc

            return lax.fori_loop(0, _HC // 16, grp, carry)

        lax.fori_loop(0, nch, chunk, 0)

        for b in range(_NW):
            w = sm[b]
            r = jnp.bitwise_and(w, 15)

            @pl.when(r != 0)
            def _tail(b=b, w=w, r=r):
                half = jnp.bitwise_and(w - r, 31)
                gb = sm[32 + b]
                for fb, ob in ((fbuf, beid_hbm), (fbuf2, bofs_hbm)):
                    hv = fb[pl.ds(pl.multiple_of(b * 32 + half, 8), 16)]
                    last = jnp.sum(jnp.where(it == r - 1, hv, 0))
                    fb[pl.ds(pl.multiple_of(b * 32 + half, 8), 16)] = jnp.where(it < r, hv, last)
                    pltpu.sync_copy(fb.at[pl.ds(pl.multiple_of(b * 32 + half, 8), 16)],
                                    ob.at[pl.ds(pl.multiple_of(gb + w - r, 8), 16)])

        total = b1[15] + t1[15]

        @pl.when(wid == _NW - 1)
        def _slack():
            zero = jnp.zeros((16,), jnp.int32)

            def zb(i, c):
                fbuf[pl.ds(i * 16, 16)] = zero
                return c

            lax.fori_loop(0, 64, zb, 0)
            pltpu.sync_copy(
                fbuf.at[pl.ds(0, _NW * 2 * _SUB)],
                beid_hbm.at[pl.ds(pl.multiple_of(total, 8), _NW * 2 * _SUB)])
            pltpu.sync_copy(
                fbuf.at[pl.ds(0, _NW * 2 * _SUB)],
                bofs_hbm.at[pl.ds(pl.multiple_of(total, 8), _NW * 2 * _SUB)])

    return bin_k


def _make_sc_scatter():
    acc_n = N * (H // 2) // _NW
    mesh = plsc.VectorSubcoreMesh(core_axis_name="c", subcore_axis_name="s")

    @functools.partial(
        pl.kernel, mesh=mesh,
        out_type=(jax.ShapeDtypeStruct((N * (H // 2),), jnp.float32),
                  jax.ShapeDtypeStruct((N * (H // 2),), jnp.float32)),
        compiler_params=pltpu.CompilerParams(use_tc_tiling_on_sc=False, needs_layout_passes=False),
        scratch_types=[pltpu.VMEM((acc_n,), jnp.float32),
                       pltpu.VMEM((acc_n,), jnp.float32),
                       pltpu.VMEM((2 * _C,), jnp.int32),
                       pltpu.VMEM((2 * _C, H), jnp.float32),
                       pltpu.VMEM((2 * _C,), jnp.int32),
                       pltpu.VMEM((_NW, _NW), jnp.int32),
                       pltpu.SemaphoreType.DMA,
                       pltpu.SemaphoreType.DMA],
    )
    def scatter_k(m_hbm, beid_hbm, bofs_hbm, cnt_hbm, alo_hbm, ahi_hbm,
                  acc_lo, acc_hi, eid_v, rows_v, ofs_v, cnt_v, sg0, sg1):
        wid = lax.axis_index("s") * 2 + lax.axis_index("c")
        pltpu.sync_copy(cnt_hbm, cnt_v)
        b0, b1, _, _, t0, t1 = _region_layout(cnt_v, wid)
        rbase = _dyn_lane(b0, b1, wid)
        cap = _dyn_lane(t0, t1, wid)

        big = jnp.full((16,), BIG, jnp.float32)

        def init(i, c):
            acc_lo[pl.ds(i * 16, 16)] = big
            acc_hi[pl.ds(i * 16, 16)] = big
            return c

        lax.fori_loop(0, acc_n // 16, init, 0)

        gsem = (sg0, sg1)

        def start(k, b):
            off = pl.multiple_of(rbase + k * _C, 8)
            pltpu.sync_copy(beid_hbm.at[pl.ds(off, _C)],
                            eid_v.at[pl.ds(b * _C, _C)])

            def cl(i, c):
                v = eid_v[pl.ds(b * _C + i * 16, 16)]
                eid_v[pl.ds(b * _C + i * 16, 16)] = jnp.clip(v, 0, E - 1)
                return c

            lax.fori_loop(0, _C // 16, cl, 0)
            pltpu.sync_copy(bofs_hbm.at[pl.ds(off, _C)],
                            ofs_v.at[pl.ds(b * _C, _C)])
            return pltpu.async_copy(m_hbm.at[eid_v.at[pl.ds(b * _C, _C)]],
                                    rows_v.at[pl.ds(b * _C, _C)], gsem[b])

        def rmw(k, b):
            valid = jnp.clip(cap - k * _C, 0, _C)

            def grp(g, cc):
                dl = jnp.clip(ofs_v[pl.ds(b * _C + g * 16, 16)],
                              0, (NB - 1) * (H // 2))
                for j in range(16):
                    o = dl[j]
                    gi = b * _C + g * 16 + j
                    r0 = rows_v[gi, pl.ds(0, 16)]
                    r1 = rows_v[gi, pl.ds(16, 16)]
                    acc_lo[pl.ds(o, 16)] = jnp.minimum(acc_lo[pl.ds(o, 16)], r0)
                    acc_hi[pl.ds(o, 16)] = jnp.minimum(acc_hi[pl.ds(o, 16)], r1)
                return cc

            lax.fori_loop(0, valid // 16, grp, 0)

        def pair(pk, c):
            k0 = 2 * pk
            cp0 = start(k0, 0)
            cp1 = start(k0 + 1, 1)
            cp0.wait()
            rmw(k0, 0)
            cp1.wait()
            rmw(k0 + 1, 1)
            return c

        npairs = (cap + 2 * _C - 1) // (2 * _C)
        lax.fori_loop(0, npairs, pair, 0)
        pltpu.sync_copy(acc_lo, alo_hbm.at[pl.ds(pl.multiple_of(wid * acc_n, 8), acc_n)])
        pltpu.sync_copy(acc_hi, ahi_hbm.at[pl.ds(pl.multiple_of(wid * acc_n, 8), acc_n)])

    return scatter_k


_sc_hist = _make_sc_hist()
_sc_bin = _make_sc_bin()
_sc_scatter = _make_sc_scatter()


def _full(shape):
    return pl.BlockSpec(shape, lambda i: (0,) * len(shape))


def _embed(x, w, b):
    return pl.pallas_call(
        _embed_body,
        grid=(N // B_N,),
        in_specs=[pl.BlockSpec((B_N, 1), lambda i: (i, 0)),
                  _full((1, H)), _full((1, H))],
        out_specs=pl.BlockSpec((B_N, H), lambda i: (i, 0)),
        out_shape=jax.ShapeDtypeStruct((N, H), jnp.float32),
    )(x, w, b)


def _edge_mlp(xj, ea, w1a, w1b, b1, w2, b2):
    return pl.pallas_call(
        _edge_mlp_body,
        grid=(E // B_E,),
        in_specs=[pl.BlockSpec((B_E, H), lambda i: (i, 0)),
                  pl.BlockSpec((B_E, ED), lambda i: (i, 0)),
                  _full((H, H)), _full((ED, H)), _full((1, H)),
                  _full((H, H)), _full((1, H))],
        out_specs=pl.BlockSpec((B_E, H), lambda i: (i, 0)),
        out_shape=jax.ShapeDtypeStruct((E, H), jnp.float32),
    )(xj, ea, w1a, w1b, b1, w2, b2)


def _node_mlp(h, alo, ahi, w1a, w1bl, w1bh, b1, w2, b2, out_dim):
    return pl.pallas_call(
        _node_mlp_body,
        grid=(N // B_N,),
        in_specs=[pl.BlockSpec((B_N, H), lambda i: (i, 0)),
                  pl.BlockSpec((B_N, H // 2), lambda i: (i, 0)),
                  pl.BlockSpec((B_N, H // 2), lambda i: (i, 0)),
                  _full((H, H)), _full((H // 2, H)), _full((H // 2, H)),
                  _full((1, H)), _full((H, out_dim)), _full((1, out_dim))],
        out_specs=pl.BlockSpec((B_N, out_dim), lambda i: (i, 0)),
        out_shape=jax.ShapeDtypeStruct((N, out_dim), jnp.float32),
    )(h, alo, ahi, w1a, w1bl, w1bh, b1, w2, b2)


def kernel(x, edge_index, edge_attr, emb_W, emb_b, msg_W1, msg_b1, msg_W2, msg_b2,
           upd_W1, upd_b1, upd_W2, upd_b2, fc_W, fc_b):
    L = msg_W1.shape[0]
    src = edge_index[0]
    dst = edge_index[1]

    cnt = _sc_hist(dst)
    beid, bofs = _sc_bin(dst, cnt)

    h = _embed(x, emb_W, emb_b.reshape(1, H))

    w2_last = upd_W2[L - 1] @ fc_W
    b2_last = (upd_b2[L - 1] @ fc_W + fc_b).reshape(1, 1)

    for l in range(L):
        xj = _sc_gather(h, src)
        m = _edge_mlp(xj, edge_attr,
                      msg_W1[l, :H], msg_W1[l, H:], msg_b1[l].reshape(1, H),
                      msg_W2[l], msg_b2[l].reshape(1, H))
        alo, ahi = _sc_scatter(m, beid, bofs, cnt)
        last = l == L - 1
        h = _node_mlp(h, alo.reshape(N, H // 2), ahi.reshape(N, H // 2),
                      upd_W1[l, :H], upd_W1[l, H:H + H // 2],
                      upd_W1[l, H + H // 2:], upd_b1[l].reshape(1, H),
                      w2_last if last else upd_W2[l],
                      b2_last if last else upd_b2[l].reshape(1, H),
                      1 if last else H)
    return h[:, 0]


# speedup vs baseline: 5.3320x; 1.7132x over previous
import functools

import jax
import jax.numpy as jnp
from jax import lax
from jax.experimental import pallas as pl
from jax.experimental.pallas import tpu as pltpu
from jax.experimental.pallas import tpu_sc as plsc

N = 100000
E = 1600000
H = 32
ED = 4

B_E = 16000
B_N = 10000

BIG = 3.0e38


def _embed_body(x_ref, w_ref, b_ref, o_ref):
    o_ref[...] = x_ref[...] @ w_ref[...] + b_ref[...]


def _edge_mlp_body(xj_ref, ea_ref, w1a_ref, w1b_ref, b1_ref, w2_ref, b2_ref, m_ref):
    z = xj_ref[...] @ w1a_ref[...] + ea_ref[...] @ w1b_ref[...] + b1_ref[...]
    m_ref[...] = jnp.maximum(z, 0.0) @ w2_ref[...] + b2_ref[...]


def _node_mlp_body(h_ref, alo_ref, ahi_ref, w1a_ref, w1bl_ref, w1bh_ref,
                   b1_ref, w2_ref, b2_ref, o_ref):
    alo = alo_ref[...]
    alo = jnp.where(alo > 1e30, 0.0, alo)
    ahi = ahi_ref[...]
    ahi = jnp.where(ahi > 1e30, 0.0, ahi)
    z = (h_ref[...] @ w1a_ref[...] + alo @ w1bl_ref[...]
         + ahi @ w1bh_ref[...] + b1_ref[...])
    o_ref[...] = jnp.maximum(z, 0.0) @ w2_ref[...] + b2_ref[...]


_NW = 32
_GC = 1000


def _make_sc_gather():
    per_w = E // _NW
    chunks = per_w // _GC

    mesh = plsc.VectorSubcoreMesh(core_axis_name="c", subcore_axis_name="s")

    @functools.partial(
        pl.kernel, mesh=mesh,
        out_type=jax.ShapeDtypeStruct((E, H), jnp.float32),
        compiler_params=pltpu.CompilerParams(use_tc_tiling_on_sc=False, needs_layout_passes=False),
        scratch_types=[
            pltpu.VMEM((2, _GC), jnp.int32),
            pltpu.VMEM((2, _GC, H), jnp.float32),
            pltpu.SemaphoreType.DMA,
            pltpu.SemaphoreType.DMA,
            pltpu.SemaphoreType.DMA,
            pltpu.SemaphoreType.DMA,
        ],
    )
    def gather_k(table_hbm, idx_hbm, out_hbm, idx_v, rows_v, s0, s1, o0, o1):
        wid = lax.axis_index("s") * 2 + lax.axis_index("c")
        base = wid * per_w
        gsem = [s0, s1]
        osem = [o0, o1]
        out_cp = [None, None]
        for i in range(chunks):
            b = i & 1
            off = base + i * _GC
            if out_cp[b] is not None:
                out_cp[b].wait()
            pltpu.sync_copy(idx_hbm.at[pl.ds(pl.multiple_of(off, 8), _GC)],
                            idx_v.at[b])
            pltpu.async_copy(table_hbm.at[idx_v.at[b]], rows_v.at[b],
                             gsem[b]).wait()
            out_cp[b] = pltpu.async_copy(
                rows_v.at[b],
                out_hbm.at[pl.ds(pl.multiple_of(off, 8), _GC)], osem[b])
        out_cp[0].wait()
        out_cp[1].wait()

    return gather_k


_sc_gather = _make_sc_gather()


NB = N // _NW
_SUB = 16
ECAP = E + _NW * _NW * _SUB + 2048
_C = 384
_HC = 2000
_PER_W = E // _NW
_INV_NB = 1.0 / NB


def _bucket_of(dv):
    return ((dv.astype(jnp.float32) + 0.5) * _INV_NB).astype(jnp.int32)


def _iota16():
    return lax.iota(jnp.int32, 16)


def _dyn_lane(v0, v1, idx):
    i = _iota16()
    lo = jnp.sum(jnp.where(i == idx, v0, 0))
    hi = jnp.sum(jnp.where(i == idx - 16, v1, 0))
    return lo + hi


def _region_layout(cnt_ref, my_p):
    z = jnp.zeros((16,), jnp.int32)

    def body(pp, carry):
        t0, t1, p0, p1 = carry
        r0 = cnt_ref[pp, pl.ds(0, 16)]
        r1 = cnt_ref[pp, pl.ds(16, 16)]
        s0 = jnp.bitwise_and(r0 + (_SUB - 1), -_SUB)
        s1 = jnp.bitwise_and(r1 + (_SUB - 1), -_SUB)
        m = pp < my_p
        return (t0 + s0, t1 + s1,
                p0 + jnp.where(m, s0, 0), p1 + jnp.where(m, s1, 0))

    t0, t1, p0, p1 = lax.fori_loop(0, _NW, body, (z, z, z, z))
    c0 = plsc.cumsum(t0)
    b0 = c0 - t0
    all0 = c0[15]
    c1 = plsc.cumsum(t1)
    b1 = c1 - t1 + all0
    return b0, b1, p0, p1, t0, t1


def _make_sc_hist():
    nch = _PER_W // _HC
    mesh = plsc.VectorSubcoreMesh(core_axis_name="c", subcore_axis_name="s")

    @functools.partial(
        pl.kernel, mesh=mesh,
        out_type=jax.ShapeDtypeStruct((_NW, _NW), jnp.int32),
        compiler_params=pltpu.CompilerParams(use_tc_tiling_on_sc=False, needs_layout_passes=False),
        scratch_types=[pltpu.VMEM((_HC,), jnp.int32),
                       pltpu.VMEM((_NW,), jnp.int32)],
    )
    def hist_k(dst_hbm, cnt_hbm, dst_v, cnt_v):
        wid = lax.axis_index("s") * 2 + lax.axis_index("c")
        base = wid * _PER_W
        it = _iota16()
        z = jnp.zeros((16,), jnp.int32)

        def chunk(k, carry):
            pltpu.sync_copy(dst_hbm.at[pl.ds(pl.multiple_of(base + k * _HC, 8), _HC)], dst_v)

            def grp(g, cc):
                d0, d1 = cc
                bv = _bucket_of(dst_v[pl.ds(g * 16, 16)])
                for j in range(16):
                    bj = bv[j]
                    d0 = d0 + jnp.where(it == bj, 1, 0)
                    d1 = d1 + jnp.where(it == bj - 16, 1, 0)
                return d0, d1

            return lax.fori_loop(0, _HC // 16, grp, carry)

        c0, c1 = lax.fori_loop(0, nch, chunk, (z, z))
        cnt_v[pl.ds(0, 16)] = c0
        cnt_v[pl.ds(16, 16)] = c1
        pltpu.sync_copy(cnt_v, cnt_hbm.at[wid])

    return hist_k


def _make_sc_bin():
    nch = _PER_W // _HC
    mesh = plsc.VectorSubcoreMesh(core_axis_name="c", subcore_axis_name="s")

    @functools.partial(
        pl.kernel, mesh=mesh,
        out_type=(jax.ShapeDtypeStruct((ECAP,), jnp.int32),
                  jax.ShapeDtypeStruct((ECAP,), jnp.int32)),
        compiler_params=pltpu.CompilerParams(use_tc_tiling_on_sc=False, needs_layout_passes=False),
        scratch_types=[pltpu.VMEM((_HC,), jnp.int32),
                       pltpu.VMEM((_NW, _NW), jnp.int32),
                       pltpu.VMEM((_NW * 2 * _SUB,), jnp.int32),
                       pltpu.VMEM((_NW * 2 * _SUB,), jnp.int32),
                       pltpu.SMEM((64,), jnp.int32)],
    )
    def bin_k(dst_hbm, cnt_hbm, beid_hbm, bofs_hbm, dst_v, cnt_v, fbuf, fbuf2, sm):
        wid = lax.axis_index("s") * 2 + lax.axis_index("c")
        base = wid * _PER_W
        pltpu.sync_copy(cnt_hbm, cnt_v)
        b0, b1, p0, p1, t0, t1 = _region_layout(cnt_v, wid)
        g0 = b0 + p0
        g1 = b1 + p1
        it = _iota16()
        for j in range(16):
            sm[j] = 0
            sm[16 + j] = 0
            sm[32 + j] = g0[j]
            sm[48 + j] = g1[j]

        def chunk(k, carry):
            pltpu.sync_copy(dst_hbm.at[pl.ds(pl.multiple_of(base + k * _HC, 8), _HC)], dst_v)

            def grp(g, cc):
                dv = dst_v[pl.ds(g * 16, 16)]
                bv = _bucket_of(dv)
                ofsv = (dv - bv * NB) * (H // 2)
                eids = base + k * _HC + g * 16 + it
                slots = jnp.zeros((16,), jnp.int32)
                post = []
                for j in range(16):
                    bj = bv[j]
                    w = sm[bj]
                    slots = jnp.where(it == j,
                                      bj * 32 + jnp.bitwise_and(w, 31), slots)
                    sm[bj] = w + 1
                    post.append((bj, w + 1))
                plsc.store_scatter(fbuf, [slots], eids)
                plsc.store_scatter(fbuf2, [slots], ofsv)
                for bj, w1 in post:
                    @pl.when(jnp.bitwise_and(w1, 15) == 0)
                    def _flush(bj=bj, w1=w1):
                        gb = sm[32 + bj]
                        half = jnp.bitwise_and(w1 - 16, 31)
                        pltpu.sync_copy(
                            fbuf.at[pl.ds(pl.multiple_of(bj * 32 + half, 8), 16)],
                            beid_hbm.at[pl.ds(pl.multiple_of(gb + w1 - 16, 8), 16)])
                        pltpu.sync_copy(
                            fbuf2.at[pl.ds(pl.multiple_of(bj * 32 + half, 8), 16)],
                            bofs_hbm.at[pl.ds(pl.multiple_of(gb + w1 - 16, 8), 16)])
                return cc

            return lax.fori_loop(0, _HC // 16, grp, carry)

        lax.fori_loop(0, nch, chunk, 0)

        for b in range(_NW):
            w = sm[b]
            r = jnp.bitwise_and(w, 15)

            @pl.when(r != 0)
            def _tail(b=b, w=w, r=r):
                half = jnp.bitwise_and(w - r, 31)
                gb = sm[32 + b]
                for fb, ob in ((fbuf, beid_hbm), (fbuf2, bofs_hbm)):
                    hv = fb[pl.ds(pl.multiple_of(b * 32 + half, 8), 16)]
                    last = jnp.sum(jnp.where(it == r - 1, hv, 0))
                    fb[pl.ds(pl.multiple_of(b * 32 + half, 8), 16)] = jnp.where(it < r, hv, last)
                    pltpu.sync_copy(fb.at[pl.ds(pl.multiple_of(b * 32 + half, 8), 16)],
                                    ob.at[pl.ds(pl.multiple_of(gb + w - r, 8), 16)])

        total = b1[15] + t1[15]

        @pl.when(wid == _NW - 1)
        def _slack():
            zero = jnp.zeros((16,), jnp.int32)

            def zb(i, c):
                fbuf[pl.ds(i * 16, 16)] = zero
                return c

            lax.fori_loop(0, 64, zb, 0)
            pltpu.sync_copy(
                fbuf.at[pl.ds(0, _NW * 2 * _SUB)],
                beid_hbm.at[pl.ds(pl.multiple_of(total, 8), _NW * 2 * _SUB)])
            pltpu.sync_copy(
                fbuf.at[pl.ds(0, _NW * 2 * _SUB)],
                bofs_hbm.at[pl.ds(pl.multiple_of(total, 8), _NW * 2 * _SUB)])

    return bin_k


def _make_sc_scatter():
    acc_n = N * (H // 2) // _NW
    mesh = plsc.VectorSubcoreMesh(core_axis_name="c", subcore_axis_name="s")

    @functools.partial(
        pl.kernel, mesh=mesh,
        out_type=(jax.ShapeDtypeStruct((N * (H // 2),), jnp.float32),
                  jax.ShapeDtypeStruct((N * (H // 2),), jnp.float32)),
        compiler_params=pltpu.CompilerParams(use_tc_tiling_on_sc=False, needs_layout_passes=False),
        scratch_types=[pltpu.VMEM((acc_n,), jnp.float32),
                       pltpu.VMEM((acc_n,), jnp.float32),
                       pltpu.VMEM((2 * _C,), jnp.int32),
                       pltpu.VMEM((2 * _C, H), jnp.float32),
                       pltpu.VMEM((2 * _C,), jnp.int32),
                       pltpu.VMEM((_NW, _NW), jnp.int32),
                       pltpu.SemaphoreType.DMA,
                       pltpu.SemaphoreType.DMA],
    )
    def scatter_k(m_hbm, beid_hbm, bofs_hbm, cnt_hbm, alo_hbm, ahi_hbm,
                  acc_lo, acc_hi, eid_v, rows_v, ofs_v, cnt_v, sg0, sg1):
        wid = lax.axis_index("s") * 2 + lax.axis_index("c")
        pltpu.sync_copy(cnt_hbm, cnt_v)
        b0, b1, _, _, t0, t1 = _region_layout(cnt_v, wid)
        rbase = _dyn_lane(b0, b1, wid)
        cap = _dyn_lane(t0, t1, wid)

        big = jnp.full((16,), BIG, jnp.float32)

        def init(i, c):
            acc_lo[pl.ds(i * 16, 16)] = big
            acc_hi[pl.ds(i * 16, 16)] = big
            return c

        lax.fori_loop(0, acc_n // 16, init, 0)

        gsem = (sg0, sg1)

        def start(k, b):
            off = pl.multiple_of(rbase + k * _C, 8)
            pltpu.sync_copy(beid_hbm.at[pl.ds(off, _C)],
                            eid_v.at[pl.ds(b * _C, _C)])

            def cl(i, c):
                v = eid_v[pl.ds(b * _C + i * 16, 16)]
                eid_v[pl.ds(b * _C + i * 16, 16)] = jnp.clip(v, 0, E - 1)
                return c

            lax.fori_loop(0, _C // 16, cl, 0)
            pltpu.sync_copy(bofs_hbm.at[pl.ds(off, _C)],
                            ofs_v.at[pl.ds(b * _C, _C)])
            return pltpu.async_copy(m_hbm.at[eid_v.at[pl.ds(b * _C, _C)]],
                                    rows_v.at[pl.ds(b * _C, _C)], gsem[b])

        def rmw(k, b):
            valid = jnp.clip(cap - k * _C, 0, _C)

            def grp(g, cc):
                dl = jnp.clip(ofs_v[pl.ds(b * _C + g * 16, 16)],
                              0, (NB - 1) * (H // 2))
                for j in range(16):
                    o = dl[j]
                    gi = b * _C + g * 16 + j
                    r0 = rows_v[gi, pl.ds(0, 16)]
                    r1 = rows_v[gi, pl.ds(16, 16)]
                    acc_lo[pl.ds(o, 16)] = jnp.minimum(acc_lo[pl.ds(o, 16)], r0)
                    acc_hi[pl.ds(o, 16)] = jnp.minimum(acc_hi[pl.ds(o, 16)], r1)
                return cc

            lax.fori_loop(0, valid // 16, grp, 0)

        def pair(pk, c):
            k0 = 2 * pk
            cp0 = start(k0, 0)
            cp1 = start(k0 + 1, 1)
            cp0.wait()
            rmw(k0, 0)
            cp1.wait()
            rmw(k0 + 1, 1)
            return c

        npairs = (cap + 2 * _C - 1) // (2 * _C)
        lax.fori_loop(0, npairs, pair, 0)
        pltpu.sync_copy(acc_lo, alo_hbm.at[pl.ds(pl.multiple_of(wid * acc_n, 8), acc_n)])
        pltpu.sync_copy(acc_hi, ahi_hbm.at[pl.ds(pl.multiple_of(wid * acc_n, 8), acc_n)])

    return scatter_k


_sc_hist = _make_sc_hist()
_sc_bin = _make_sc_bin()
_sc_scatter = _make_sc_scatter()


def _full(shape):
    return pl.BlockSpec(shape, lambda i: (0,) * len(shape))


def _embed(x, w, b):
    return pl.pallas_call(
        _embed_body,
        grid=(N // B_N,),
        in_specs=[pl.BlockSpec((B_N, 1), lambda i: (i, 0)),
                  _full((1, H)), _full((1, H))],
        out_specs=pl.BlockSpec((B_N, H), lambda i: (i, 0)),
        out_shape=jax.ShapeDtypeStruct((N, H), jnp.float32),
    )(x, w, b)


def _edge_mlp(xj4, ea4, w1a4, w1b4, b1_4, w2_4, b2_4):
    e4 = E // 4
    b4 = B_E // 4
    return pl.pallas_call(
        _edge_mlp_body,
        grid=(e4 // b4,),
        in_specs=[pl.BlockSpec((b4, 4 * H), lambda i: (i, 0)),
                  pl.BlockSpec((b4, 4 * ED), lambda i: (i, 0)),
                  _full((4 * H, 4 * H)), _full((4 * ED, 4 * H)),
                  _full((1, 4 * H)),
                  _full((4 * H, 4 * H)), _full((1, 4 * H))],
        out_specs=pl.BlockSpec((b4, 4 * H), lambda i: (i, 0)),
        out_shape=jax.ShapeDtypeStruct((e4, 4 * H), jnp.float32),
    )(xj4, ea4, w1a4, w1b4, b1_4, w2_4, b2_4)


def _blkdiag4(w):
    a, b = w.shape
    z = jnp.zeros((4 * a, 4 * b), w.dtype)
    for q in range(4):
        z = z.at[q * a:(q + 1) * a, q * b:(q + 1) * b].set(w)
    return z


def _node_mlp(h, alo, ahi, w1a, w1bl, w1bh, b1, w2, b2, out_dim):
    return pl.pallas_call(
        _node_mlp_body,
        grid=(N // B_N,),
        in_specs=[pl.BlockSpec((B_N, H), lambda i: (i, 0)),
                  pl.BlockSpec((B_N, H // 2), lambda i: (i, 0)),
                  pl.BlockSpec((B_N, H // 2), lambda i: (i, 0)),
                  _full((H, H)), _full((H // 2, H)), _full((H // 2, H)),
                  _full((1, H)), _full((H, out_dim)), _full((1, out_dim))],
        out_specs=pl.BlockSpec((B_N, out_dim), lambda i: (i, 0)),
        out_shape=jax.ShapeDtypeStruct((N, out_dim), jnp.float32),
    )(h, alo, ahi, w1a, w1bl, w1bh, b1, w2, b2)


def kernel(x, edge_index, edge_attr, emb_W, emb_b, msg_W1, msg_b1, msg_W2, msg_b2,
           upd_W1, upd_b1, upd_W2, upd_b2, fc_W, fc_b):
    L = msg_W1.shape[0]
    src = edge_index[0]
    dst = edge_index[1]

    cnt = _sc_hist(dst)
    beid, bofs = _sc_bin(dst, cnt)

    h = _embed(x, emb_W, emb_b.reshape(1, H))

    w2_last = upd_W2[L - 1] @ fc_W
    b2_last = (upd_b2[L - 1] @ fc_W + fc_b).reshape(1, 1)

    ea4 = edge_attr.reshape(E // 4, 4 * ED)
    for l in range(L):
        xj4 = _sc_gather(h, src).reshape(E // 4, 4 * H)
        m4 = _edge_mlp(xj4, ea4,
                       _blkdiag4(msg_W1[l, :H]), _blkdiag4(msg_W1[l, H:]),
                       jnp.tile(msg_b1[l], 4).reshape(1, 4 * H),
                       _blkdiag4(msg_W2[l]),
                       jnp.tile(msg_b2[l], 4).reshape(1, 4 * H))
        m = m4.reshape(E, H)
        alo, ahi = _sc_scatter(m, beid, bofs, cnt)
        last = l == L - 1
        h = _node_mlp(h, alo.reshape(N, H // 2), ahi.reshape(N, H // 2),
                      upd_W1[l, :H], upd_W1[l, H:H + H // 2],
                      upd_W1[l, H + H // 2:], upd_b1[l].reshape(1, H),
                      w2_last if last else upd_W2[l],
                      b2_last if last else upd_b2[l].reshape(1, H),
                      1 if last else H)
    return h[:, 0]
